# Initial kernel scaffold; baseline (speedup 1.0000x reference)
import jax
import jax.numpy as jnp
from jax.experimental import pallas as pl


def kernel(states, next_states, dissimilar_pairs, same_actions_pairs, ref_point_pairs, similar_pairs, W):
    raise NotImplementedError("write your pallas kernel here")


# speedup vs baseline: 1.0120x; 1.0120x over previous
import functools

import jax
import jax.numpy as jnp
from jax import lax
from jax.experimental import pallas as pl
from jax.experimental.pallas import tpu as pltpu
from jax.experimental.pallas import tpu_sc as plsc

_N = 65536
_D = 64
_P = 65536
_R = 16384
_L1_COEFF = 0.001 / float(_D * _D)

_NC = 2
_NS = 16
_NW = _NC * _NS
_CH = 128
_LANES = 16


def _sqrt16(x):
    i = lax.bitcast_convert_type(x, jnp.int32)
    i = jnp.int32(0x1FBD1DF5) + lax.shift_right_logical(i, 1)
    y = lax.bitcast_convert_type(i, jnp.float32)
    for _ in range(3):
        y = 0.5 * (y + x / y)
    return y


@functools.partial(
    pl.kernel,
    mesh=plsc.VectorSubcoreMesh(core_axis_name="c", subcore_axis_name="s"),
    compiler_params=pltpu.CompilerParams(
        needs_layout_passes=False, use_tc_tiling_on_sc=False),
    out_type=jax.ShapeDtypeStruct((_NW, 4 * _LANES), jnp.float32),
    scratch_types=[
        pltpu.VMEM((_CH,), jnp.int32),
        pltpu.VMEM((_CH,), jnp.int32),
        pltpu.VMEM((_CH, _D), jnp.float32),
        pltpu.VMEM((_CH, _D), jnp.float32),
        pltpu.VMEM((_CH, _D), jnp.float32),
        pltpu.VMEM((_CH, _D), jnp.float32),
        pltpu.VMEM((_LANES * _LANES,), jnp.float32),
        pltpu.VMEM((_LANES * _LANES,), jnp.float32),
        pltpu.VMEM((_LANES * _LANES,), jnp.float32),
        pltpu.VMEM((_LANES * _LANES,), jnp.float32),
        pltpu.VMEM((4 * _LANES,), jnp.float32),
        pltpu.SemaphoreType.DMA,
    ],
)
def _sc_pair_losses(states_hbm, nstates_hbm, disa_hbm, disb_hbm, saa_hbm,
                    sab_hbm, refa_hbm, refb_hbm, out_hbm,
                    ia_v, ib_v, bufa, bufb, bufc, bufd,
                    fold1, fold2, fold3, fold4, accs, sem):
    wid = lax.axis_index("s") * _NC + lax.axis_index("c")
    lane = lax.iota(jnp.int32, _LANES)
    zero = jnp.zeros((_LANES,), jnp.float32)

    def fetch_pairs(a_hbm, b_hbm, base):
        pltpu.sync_copy(a_hbm.at[pl.ds(base, _CH)], ia_v)
        pltpu.sync_copy(b_hbm.at[pl.ds(base, _CH)], ib_v)

    def transpose_sum(fold):
        n2 = zero
        for j in range(_LANES):
            n2 = n2 + plsc.load_gather(fold, [lane * _LANES + j])
        return n2

    def dist2(gbase, b1, b2):
        for p in range(_LANES):
            acc = None
            for k in range(_D // _LANES):
                va = b1[gbase + p, pl.ds(k * _LANES, _LANES)]
                vb = b2[gbase + p, pl.ds(k * _LANES, _LANES)]
                dv = va - vb
                acc = dv * dv if acc is None else acc + dv * dv
            fold1[pl.ds(p * _LANES, _LANES)] = acc
        return transpose_sum(fold1)

    n_chunks = (_P // _NW) // _CH

    def dis_chunk(c, acc):
        base = wid * (_P // _NW) + c * _CH
        fetch_pairs(disa_hbm, disb_hbm, base)
        pltpu.async_copy(states_hbm.at[ia_v], bufa, sem).wait()
        pltpu.async_copy(states_hbm.at[ib_v], bufb, sem).wait()

        def grp(g, a):
            n2 = dist2(g * _LANES, bufa, bufb)
            return a + jnp.exp(-n2)

        return lax.fori_loop(0, _CH // _LANES, grp, acc)

    acc_caus = lax.fori_loop(0, n_chunks, dis_chunk, zero)

    def sa_chunk(c, carry):
        accp, accr = carry
        base = wid * (_P // _NW) + c * _CH
        fetch_pairs(saa_hbm, sab_hbm, base)
        pltpu.async_copy(states_hbm.at[ia_v], bufa, sem).wait()
        pltpu.async_copy(states_hbm.at[ib_v], bufb, sem).wait()
        pltpu.async_copy(nstates_hbm.at[ia_v], bufc, sem).wait()
        pltpu.async_copy(nstates_hbm.at[ib_v], bufd, sem).wait()

        def grp(g, cr):
            ap, ar = cr
            gbase = g * _LANES
            for p in range(_LANES):
                f1 = f2 = f3 = f4 = None
                for k in range(_D // _LANES):
                    sl = pl.ds(k * _LANES, _LANES)
                    sa_ = bufa[gbase + p, sl]
                    sb_ = bufb[gbase + p, sl]
                    na_ = bufc[gbase + p, sl]
                    nb_ = bufd[gbase + p, sl]
                    ds = sa_ - sb_
                    da = na_ - sa_
                    db = nb_ - sb_
                    dd = da - db
                    if f1 is None:
                        f1, f2, f3, f4 = ds * ds, dd * dd, da * da, db * db
                    else:
                        f1 = f1 + ds * ds
                        f2 = f2 + dd * dd
                        f3 = f3 + da * da
                        f4 = f4 + db * db
                psl = pl.ds(p * _LANES, _LANES)
                fold1[psl] = f1
                fold2[psl] = f2
                fold3[psl] = f3
                fold4[psl] = f4
            n2s = transpose_sum(fold1)
            n2d = transpose_sum(fold2)
            n2a = transpose_sum(fold3)
            n2b = transpose_sum(fold4)
            dsn = _sqrt16(n2a) - _sqrt16(n2b)
            ap = ap + dsn * dsn
            ar = ar + jnp.exp(-n2s) * n2d
            return (ap, ar)

        return lax.fori_loop(0, _CH // _LANES, grp, (accp, accr))

    acc_prop, acc_rep = lax.fori_loop(0, n_chunks, sa_chunk, (zero, zero))

    def ref_chunk(c, acc):
        base = wid * (_R // _NW) + c * _CH
        fetch_pairs(refa_hbm, refb_hbm, base)
        pltpu.async_copy(states_hbm.at[ia_v], bufa, sem).wait()
        pltpu.async_copy(states_hbm.at[ib_v], bufb, sem).wait()

        def grp(g, a):
            return a + dist2(g * _LANES, bufa, bufb)

        return lax.fori_loop(0, _CH // _LANES, grp, acc)

    acc_fix = lax.fori_loop(0, (_R // _NW) // _CH, ref_chunk, zero)

    accs[pl.ds(0, _LANES)] = acc_caus
    accs[pl.ds(_LANES, _LANES)] = acc_prop
    accs[pl.ds(2 * _LANES, _LANES)] = acc_rep
    accs[pl.ds(3 * _LANES, _LANES)] = acc_fix
    pltpu.sync_copy(accs, out_hbm.at[wid])


_TBLK = 1024


def _tc_body(s_ref, ns_ref, w_ref, part_ref):
    d = ns_ref[...] - s_ref[...]
    tot = jnp.sum(d * d)
    wsum = jnp.sum(jnp.abs(w_ref[...]))
    lanes = lax.broadcasted_iota(jnp.int32, (1, 8, 128), 2)
    part_ref[...] = jnp.where(lanes == 0, tot, jnp.where(lanes == 1, wsum, 0.0))


_tc_dense = pl.pallas_call(
    _tc_body,
    grid=(_N // _TBLK,),
    in_specs=[
        pl.BlockSpec((_TBLK, _D), lambda i: (i, 0)),
        pl.BlockSpec((_TBLK, _D), lambda i: (i, 0)),
        pl.BlockSpec((_D, _D), lambda i: (0, 0)),
    ],
    out_specs=pl.BlockSpec((1, 8, 128), lambda i: (i, 0, 0)),
    out_shape=jax.ShapeDtypeStruct((_N // _TBLK, 8, 128), jnp.float32),
)


def kernel(states, next_states, dissimilar_pairs, same_actions_pairs,
           ref_point_pairs, similar_pairs, W):
    del similar_pairs
    part = _tc_dense(states, next_states, W)
    sc_part = _sc_pair_losses(
        states, next_states,
        dissimilar_pairs[:, 0], dissimilar_pairs[:, 1],
        same_actions_pairs[:, 0], same_actions_pairs[:, 1],
        ref_point_pairs[:, 0], ref_point_pairs[:, 1],
    )
    sums = jnp.sum(sc_part.reshape(_NW, 4, _LANES), axis=(0, 2))
    temp_coherence = jnp.sum(part[:, 0, 0]) / _N
    l1 = part[0, 0, 1]
    return (temp_coherence
            + sums[0] / _P
            + sums[1] / _P
            + sums[2] / _P
            + sums[3] / _R
            + _L1_COEFF * l1)


# speedup vs baseline: 1.4937x; 1.4759x over previous
import functools

import jax
import jax.numpy as jnp
from jax import lax
from jax.experimental import pallas as pl
from jax.experimental.pallas import tpu as pltpu
from jax.experimental.pallas import tpu_sc as plsc

_N = 65536
_D = 64
_P = 65536
_R = 16384
_L1_COEFF = 0.001 / float(_D * _D)

_NC = 2
_NS = 16
_NW = _NC * _NS
_CH = 128
_LANES = 16


def _sqrt16(x):
    i = lax.bitcast_convert_type(x, jnp.int32)
    i = jnp.int32(0x1FBD1DF5) + lax.shift_right_logical(i, 1)
    y = lax.bitcast_convert_type(i, jnp.float32)
    for _ in range(3):
        y = 0.5 * (y + x / y)
    return y


@functools.partial(
    pl.kernel,
    mesh=plsc.VectorSubcoreMesh(core_axis_name="c", subcore_axis_name="s"),
    compiler_params=pltpu.CompilerParams(
        needs_layout_passes=False, use_tc_tiling_on_sc=False),
    out_type=jax.ShapeDtypeStruct((_NW, 4 * _LANES), jnp.float32),
    scratch_types=[
        pltpu.VMEM((_P // _NW,), jnp.int32),
        pltpu.VMEM((_P // _NW,), jnp.int32),
        pltpu.VMEM((_CH, _D), jnp.float32),
        pltpu.VMEM((_CH, _D), jnp.float32),
        pltpu.VMEM((_CH, _D), jnp.float32),
        pltpu.VMEM((_CH, _D), jnp.float32),
        pltpu.VMEM((_CH, _D), jnp.float32),
        pltpu.VMEM((_CH, _D), jnp.float32),
        pltpu.VMEM((_CH, _D), jnp.float32),
        pltpu.VMEM((_CH, _D), jnp.float32),
        pltpu.VMEM((_LANES * _LANES,), jnp.float32),
        pltpu.VMEM((_LANES * _LANES,), jnp.float32),
        pltpu.VMEM((_LANES * _LANES,), jnp.float32),
        pltpu.VMEM((_LANES * _LANES,), jnp.float32),
        pltpu.VMEM((4 * _LANES,), jnp.float32),
        pltpu.SemaphoreType.DMA,
        pltpu.SemaphoreType.DMA,
    ],
)
def _sc_pair_losses(states_hbm, nstates_hbm, disa_hbm, disb_hbm, saa_hbm,
                    sab_hbm, refa_hbm, refb_hbm, out_hbm,
                    ia_all, ib_all, bufa0, bufa1, bufb0, bufb1,
                    bufc0, bufc1, bufd0, bufd1,
                    fold1, fold2, fold3, fold4, accs, sem0, sem1):
    wid = lax.axis_index("s") * _NC + lax.axis_index("c")
    lane = lax.iota(jnp.int32, _LANES)
    zero = jnp.zeros((_LANES,), jnp.float32)
    bufa = (bufa0, bufa1)
    bufb = (bufb0, bufb1)
    bufc = (bufc0, bufc1)
    bufd = (bufd0, bufd1)
    sems = (sem0, sem1)

    def transpose_sum(fold):
        n2 = zero
        for j in range(_LANES):
            n2 = n2 + plsc.load_gather(fold, [lane * _LANES + j])
        return n2

    def dist2(gbase, b1, b2):
        for p in range(_LANES):
            acc = None
            for k in range(_D // _LANES):
                va = b1[gbase + p, pl.ds(k * _LANES, _LANES)]
                vb = b2[gbase + p, pl.ds(k * _LANES, _LANES)]
                dv = va - vb
                acc = dv * dv if acc is None else acc + dv * dv
            fold1[pl.ds(p * _LANES, _LANES)] = acc
        return transpose_sum(fold1)

    def load_idx(a_hbm, b_hbm, per_w):
        pltpu.sync_copy(a_hbm.at[pl.ds(wid * per_w, per_w)],
                        ia_all.at[pl.ds(0, per_w)])
        pltpu.sync_copy(b_hbm.at[pl.ds(wid * per_w, per_w)],
                        ib_all.at[pl.ds(0, per_w)])

    def fire2(c, s):
        ia = ia_all.at[pl.ds(c * _CH, _CH)]
        ib = ib_all.at[pl.ds(c * _CH, _CH)]
        pltpu.async_copy(states_hbm.at[ia], bufa[s], sems[s])
        pltpu.async_copy(states_hbm.at[ib], bufb[s], sems[s])

    def drain2(s):
        ia = ia_all.at[pl.ds(0, _CH)]
        pltpu.make_async_copy(states_hbm.at[ia], bufa[s], sems[s]).wait()
        pltpu.make_async_copy(states_hbm.at[ia], bufb[s], sems[s]).wait()

    def fire4(c, s):
        ia = ia_all.at[pl.ds(c * _CH, _CH)]
        ib = ib_all.at[pl.ds(c * _CH, _CH)]
        pltpu.async_copy(states_hbm.at[ia], bufa[s], sems[s])
        pltpu.async_copy(states_hbm.at[ib], bufb[s], sems[s])
        pltpu.async_copy(nstates_hbm.at[ia], bufc[s], sems[s])
        pltpu.async_copy(nstates_hbm.at[ib], bufd[s], sems[s])

    def drain4(s):
        ia = ia_all.at[pl.ds(0, _CH)]
        pltpu.make_async_copy(states_hbm.at[ia], bufa[s], sems[s]).wait()
        pltpu.make_async_copy(states_hbm.at[ia], bufb[s], sems[s]).wait()
        pltpu.make_async_copy(states_hbm.at[ia], bufc[s], sems[s]).wait()
        pltpu.make_async_copy(states_hbm.at[ia], bufd[s], sems[s]).wait()

    def pipelined(n, fire, drain, compute, carry):
        fire(0, 0)
        fire(1, 1)

        def body(k, cr):
            c = 2 * k
            drain(0)
            cr = compute(0, c, cr)
            fire(c + 2, 0)
            drain(1)
            cr = compute(1, c + 1, cr)
            fire(c + 3, 1)
            return cr

        carry = lax.fori_loop(0, n // 2 - 1, body, carry)
        drain(0)
        carry = compute(0, n - 2, carry)
        drain(1)
        carry = compute(1, n - 1, carry)
        return carry

    n_chunks = (_P // _NW) // _CH

    def dis_compute(s, c, acc):
        def grp(g, a):
            n2 = dist2(g * _LANES, bufa[s], bufb[s])
            return a + jnp.exp(-n2)

        return lax.fori_loop(0, _CH // _LANES, grp, acc)

    load_idx(disa_hbm, disb_hbm, _P // _NW)
    acc_caus = pipelined(n_chunks, fire2, drain2, dis_compute, zero)

    def sa_compute(s, c, carry):
        ba, bb, bc, bd = bufa[s], bufb[s], bufc[s], bufd[s]

        def grp(g, cr):
            ap, ar = cr
            gbase = g * _LANES
            for p in range(_LANES):
                f1 = f2 = f3 = f4 = None
                for k in range(_D // _LANES):
                    sl = pl.ds(k * _LANES, _LANES)
                    sa_ = ba[gbase + p, sl]
                    sb_ = bb[gbase + p, sl]
                    na_ = bc[gbase + p, sl]
                    nb_ = bd[gbase + p, sl]
                    ds = sa_ - sb_
                    da = na_ - sa_
                    db = nb_ - sb_
                    dd = da - db
                    if f1 is None:
                        f1, f2, f3, f4 = ds * ds, dd * dd, da * da, db * db
                    else:
                        f1 = f1 + ds * ds
                        f2 = f2 + dd * dd
                        f3 = f3 + da * da
                        f4 = f4 + db * db
                psl = pl.ds(p * _LANES, _LANES)
                fold1[psl] = f1
                fold2[psl] = f2
                fold3[psl] = f3
                fold4[psl] = f4
            n2s = transpose_sum(fold1)
            n2d = transpose_sum(fold2)
            n2a = transpose_sum(fold3)
            n2b = transpose_sum(fold4)
            dsn = _sqrt16(n2a) - _sqrt16(n2b)
            ap = ap + dsn * dsn
            ar = ar + jnp.exp(-n2s) * n2d
            return (ap, ar)

        return lax.fori_loop(0, _CH // _LANES, grp, carry)

    load_idx(saa_hbm, sab_hbm, _P // _NW)
    acc_prop, acc_rep = pipelined(n_chunks, fire4, drain4, sa_compute,
                                  (zero, zero))

    def ref_compute(s, c, acc):
        def grp(g, a):
            return a + dist2(g * _LANES, bufa[s], bufb[s])

        return lax.fori_loop(0, _CH // _LANES, grp, acc)

    load_idx(refa_hbm, refb_hbm, _R // _NW)
    acc_fix = pipelined((_R // _NW) // _CH, fire2, drain2, ref_compute, zero)

    accs[pl.ds(0, _LANES)] = acc_caus
    accs[pl.ds(_LANES, _LANES)] = acc_prop
    accs[pl.ds(2 * _LANES, _LANES)] = acc_rep
    accs[pl.ds(3 * _LANES, _LANES)] = acc_fix
    pltpu.sync_copy(accs, out_hbm.at[wid])


_TBLK = 1024


def _tc_body(s_ref, ns_ref, w_ref, part_ref):
    d = ns_ref[...] - s_ref[...]
    tot = jnp.sum(d * d)
    wsum = jnp.sum(jnp.abs(w_ref[...]))
    lanes = lax.broadcasted_iota(jnp.int32, (1, 8, 128), 2)
    part_ref[...] = jnp.where(lanes == 0, tot, jnp.where(lanes == 1, wsum, 0.0))


_tc_dense = pl.pallas_call(
    _tc_body,
    grid=(_N // _TBLK,),
    in_specs=[
        pl.BlockSpec((_TBLK, _D), lambda i: (i, 0)),
        pl.BlockSpec((_TBLK, _D), lambda i: (i, 0)),
        pl.BlockSpec((_D, _D), lambda i: (0, 0)),
    ],
    out_specs=pl.BlockSpec((1, 8, 128), lambda i: (i, 0, 0)),
    out_shape=jax.ShapeDtypeStruct((_N // _TBLK, 8, 128), jnp.float32),
)


def kernel(states, next_states, dissimilar_pairs, same_actions_pairs,
           ref_point_pairs, similar_pairs, W):
    del similar_pairs
    part = _tc_dense(states, next_states, W)
    sc_part = _sc_pair_losses(
        states, next_states,
        dissimilar_pairs[:, 0], dissimilar_pairs[:, 1],
        same_actions_pairs[:, 0], same_actions_pairs[:, 1],
        ref_point_pairs[:, 0], ref_point_pairs[:, 1],
    )
    sums = jnp.sum(sc_part.reshape(_NW, 4, _LANES), axis=(0, 2))
    temp_coherence = jnp.sum(part[:, 0, 0]) / _N
    l1 = part[0, 0, 1]
    return (temp_coherence
            + sums[0] / _P
            + sums[1] / _P
            + sums[2] / _P
            + sums[3] / _R
            + _L1_COEFF * l1)


# speedup vs baseline: 1.8541x; 1.2413x over previous
import functools

import jax
import jax.numpy as jnp
from jax import lax
from jax.experimental import pallas as pl
from jax.experimental.pallas import tpu as pltpu
from jax.experimental.pallas import tpu_sc as plsc

_N = 65536
_D = 64
_P = 65536
_R = 16384
_L1_COEFF = 0.001 / float(_D * _D)

_NC = 2
_NS = 16
_NW = _NC * _NS
_CH = 128
_LANES = 16

_SC_PARAMS = pltpu.CompilerParams(
    needs_layout_passes=False, use_tc_tiling_on_sc=False)
_SC_MESH = dict(core_axis_name="c", subcore_axis_name="s")


def _sqrt16(x):
    i = lax.bitcast_convert_type(x, jnp.int32)
    i = jnp.int32(0x1FBD1DF5) + lax.shift_right_logical(i, 1)
    y = lax.bitcast_convert_type(i, jnp.float32)
    for _ in range(3):
        y = 0.5 * (y + x / y)
    return y


def _transpose_sum(fold, lane):
    n2 = None
    for j in range(_LANES):
        v = plsc.load_gather(fold, [lane * _LANES + j])
        n2 = v if n2 is None else n2 + v
    return n2


def _pipelined(n, fire, drain, compute, carry):
    fire(0, 0)
    fire(1, 1)

    def body(k, cr):
        c = 2 * k
        drain(0)
        cr = compute(0, c, cr)
        fire(c + 2, 0)
        drain(1)
        cr = compute(1, c + 1, cr)
        fire(c + 3, 1)
        return cr

    carry = lax.fori_loop(0, n // 2 - 1, body, carry)
    drain(0)
    carry = compute(0, n - 2, carry)
    drain(1)
    carry = compute(1, n - 1, carry)
    return carry


@functools.partial(
    pl.kernel,
    mesh=plsc.VectorSubcoreMesh(**_SC_MESH),
    compiler_params=_SC_PARAMS,
    out_type=jax.ShapeDtypeStruct((_NW, 2 * _LANES), jnp.float32),
    scratch_types=[
        pltpu.VMEM((_P // _NW,), jnp.int32),
        pltpu.VMEM((_P // _NW,), jnp.int32),
        pltpu.VMEM((_CH, _D), jnp.float32),
        pltpu.VMEM((_CH, _D), jnp.float32),
        pltpu.VMEM((_CH, _D), jnp.float32),
        pltpu.VMEM((_CH, _D), jnp.float32),
        pltpu.VMEM((_LANES * _LANES,), jnp.float32),
        pltpu.VMEM((2 * _LANES,), jnp.float32),
        pltpu.SemaphoreType.DMA,
        pltpu.SemaphoreType.DMA,
    ],
)
def _sc_states_losses(states_hbm, disa_hbm, disb_hbm, refa_hbm, refb_hbm,
                      out_hbm, ia_all, ib_all, bufa0, bufa1, bufb0, bufb1,
                      fold1, accs, sem0, sem1):
    wid = lax.axis_index("s") * _NC + lax.axis_index("c")
    lane = lax.iota(jnp.int32, _LANES)
    zero = jnp.zeros((_LANES,), jnp.float32)
    bufa = (bufa0, bufa1)
    bufb = (bufb0, bufb1)
    sems = (sem0, sem1)

    def load_idx(a_hbm, b_hbm, per_w):
        pltpu.sync_copy(a_hbm.at[pl.ds(wid * per_w, per_w)],
                        ia_all.at[pl.ds(0, per_w)])
        pltpu.sync_copy(b_hbm.at[pl.ds(wid * per_w, per_w)],
                        ib_all.at[pl.ds(0, per_w)])

    def fire(c, s):
        ia = ia_all.at[pl.ds(c * _CH, _CH)]
        ib = ib_all.at[pl.ds(c * _CH, _CH)]
        pltpu.async_copy(states_hbm.at[ia], bufa[s], sems[s])
        pltpu.async_copy(states_hbm.at[ib], bufb[s], sems[s])

    def drain(s):
        ia = ia_all.at[pl.ds(0, _CH)]
        pltpu.make_async_copy(states_hbm.at[ia], bufa[s], sems[s]).wait()
        pltpu.make_async_copy(states_hbm.at[ia], bufb[s], sems[s]).wait()

    def dist2(gbase, b1, b2):
        for p in range(_LANES):
            acc = None
            for k in range(_D // _LANES):
                va = b1[gbase + p, pl.ds(k * _LANES, _LANES)]
                vb = b2[gbase + p, pl.ds(k * _LANES, _LANES)]
                dv = va - vb
                acc = dv * dv if acc is None else acc + dv * dv
            fold1[pl.ds(p * _LANES, _LANES)] = acc
        return _transpose_sum(fold1, lane)

    def dis_compute(s, c, acc):
        def grp(g, a):
            return a + jnp.exp(-dist2(g * _LANES, bufa[s], bufb[s]))

        return lax.fori_loop(0, _CH // _LANES, grp, acc)

    def ref_compute(s, c, acc):
        def grp(g, a):
            return a + dist2(g * _LANES, bufa[s], bufb[s])

        return lax.fori_loop(0, _CH // _LANES, grp, acc)

    load_idx(disa_hbm, disb_hbm, _P // _NW)
    acc_caus = _pipelined((_P // _NW) // _CH, fire, drain, dis_compute, zero)
    load_idx(refa_hbm, refb_hbm, _R // _NW)
    acc_fix = _pipelined((_R // _NW) // _CH, fire, drain, ref_compute, zero)

    accs[pl.ds(0, _LANES)] = acc_caus
    accs[pl.ds(_LANES, _LANES)] = acc_fix
    pltpu.sync_copy(accs, out_hbm.at[wid])


@functools.partial(
    pl.kernel,
    mesh=plsc.VectorSubcoreMesh(**_SC_MESH),
    compiler_params=_SC_PARAMS,
    out_type=jax.ShapeDtypeStruct((_NW, 2 * _LANES), jnp.float32),
    scratch_types=[
        pltpu.VMEM((_P // _NW,), jnp.int32),
        pltpu.VMEM((_P // _NW,), jnp.int32),
        pltpu.VMEM((_CH, _D), jnp.float32),
        pltpu.VMEM((_CH, _D), jnp.float32),
        pltpu.VMEM((_CH, _D), jnp.float32),
        pltpu.VMEM((_CH, _D), jnp.float32),
        pltpu.VMEM((_CH, _D), jnp.float32),
        pltpu.VMEM((_CH, _D), jnp.float32),
        pltpu.VMEM((_CH, _D), jnp.float32),
        pltpu.VMEM((_CH, _D), jnp.float32),
        pltpu.VMEM((_LANES * _LANES,), jnp.float32),
        pltpu.VMEM((_LANES * _LANES,), jnp.float32),
        pltpu.VMEM((_LANES * _LANES,), jnp.float32),
        pltpu.VMEM((_LANES * _LANES,), jnp.float32),
        pltpu.VMEM((2 * _LANES,), jnp.float32),
        pltpu.SemaphoreType.DMA,
        pltpu.SemaphoreType.DMA,
    ],
)
def _sc_pairdiff_losses(states_hbm, nstates_hbm, saa_hbm, sab_hbm,
                        out_hbm, ia_all, ib_all, bufa0, bufa1, bufb0, bufb1,
                        bufc0, bufc1, bufd0, bufd1,
                        fold1, fold2, fold3, fold4, accs, sem0, sem1):
    wid = lax.axis_index("s") * _NC + lax.axis_index("c")
    lane = lax.iota(jnp.int32, _LANES)
    zero = jnp.zeros((_LANES,), jnp.float32)
    bufa = (bufa0, bufa1)
    bufb = (bufb0, bufb1)
    bufc = (bufc0, bufc1)
    bufd = (bufd0, bufd1)
    sems = (sem0, sem1)
    per_w = _P // _NW

    pltpu.sync_copy(saa_hbm.at[pl.ds(wid * per_w, per_w)], ia_all)
    pltpu.sync_copy(sab_hbm.at[pl.ds(wid * per_w, per_w)], ib_all)

    def fire(c, s):
        ia = ia_all.at[pl.ds(c * _CH, _CH)]
        ib = ib_all.at[pl.ds(c * _CH, _CH)]
        pltpu.async_copy(states_hbm.at[ia], bufa[s], sems[s])
        pltpu.async_copy(states_hbm.at[ib], bufb[s], sems[s])
        pltpu.async_copy(nstates_hbm.at[ia], bufc[s], sems[s])
        pltpu.async_copy(nstates_hbm.at[ib], bufd[s], sems[s])

    def drain(s):
        ia = ia_all.at[pl.ds(0, _CH)]
        pltpu.make_async_copy(states_hbm.at[ia], bufa[s], sems[s]).wait()
        pltpu.make_async_copy(states_hbm.at[ia], bufb[s], sems[s]).wait()
        pltpu.make_async_copy(states_hbm.at[ia], bufc[s], sems[s]).wait()
        pltpu.make_async_copy(states_hbm.at[ia], bufd[s], sems[s]).wait()

    def sa_compute(s, c, carry):
        ba, bb, bc, bd = bufa[s], bufb[s], bufc[s], bufd[s]

        def grp(g, cr):
            ap, ar = cr
            gbase = g * _LANES
            for p in range(_LANES):
                f1 = f2 = f3 = f4 = None
                for k in range(_D // _LANES):
                    sl = pl.ds(k * _LANES, _LANES)
                    sa_ = ba[gbase + p, sl]
                    sb_ = bb[gbase + p, sl]
                    na_ = bc[gbase + p, sl]
                    nb_ = bd[gbase + p, sl]
                    ds = sa_ - sb_
                    da = na_ - sa_
                    db = nb_ - sb_
                    dd = da - db
                    if f1 is None:
                        f1, f2, f3, f4 = ds * ds, dd * dd, da * da, db * db
                    else:
                        f1 = f1 + ds * ds
                        f2 = f2 + dd * dd
                        f3 = f3 + da * da
                        f4 = f4 + db * db
                psl = pl.ds(p * _LANES, _LANES)
                fold1[psl] = f1
                fold2[psl] = f2
                fold3[psl] = f3
                fold4[psl] = f4
            n2s = _transpose_sum(fold1, lane)
            n2d = _transpose_sum(fold2, lane)
            n2a = _transpose_sum(fold3, lane)
            n2b = _transpose_sum(fold4, lane)
            dsn = _sqrt16(n2a) - _sqrt16(n2b)
            ap = ap + dsn * dsn
            ar = ar + jnp.exp(-n2s) * n2d
            return (ap, ar)

        return lax.fori_loop(0, _CH // _LANES, grp, carry)

    acc_prop, acc_rep = _pipelined(per_w // _CH, fire, drain, sa_compute,
                                   (zero, zero))

    accs[pl.ds(0, _LANES)] = acc_prop
    accs[pl.ds(_LANES, _LANES)] = acc_rep
    pltpu.sync_copy(accs, out_hbm.at[wid])


_TBLK = 131072


def _tc_body(s_ref, ns_ref, w_ref, part_ref):
    d = ns_ref[...] - s_ref[...]
    tot = jnp.sum(d * d)
    wsum = jnp.sum(jnp.abs(w_ref[...]))
    lanes = lax.broadcasted_iota(jnp.int32, (1, 8, 128), 2)
    part_ref[...] = jnp.where(lanes == 0, tot, jnp.where(lanes == 1, wsum, 0.0))


_tc_dense = pl.pallas_call(
    _tc_body,
    grid=(_N * _D // _TBLK,),
    in_specs=[
        pl.BlockSpec((_TBLK,), lambda i: (i,)),
        pl.BlockSpec((_TBLK,), lambda i: (i,)),
        pl.BlockSpec((_D, _D), lambda i: (0, 0)),
    ],
    out_specs=pl.BlockSpec((1, 8, 128), lambda i: (i, 0, 0)),
    out_shape=jax.ShapeDtypeStruct((_N * _D // _TBLK, 8, 128), jnp.float32),
)


def kernel(states, next_states, dissimilar_pairs, same_actions_pairs,
           ref_point_pairs, similar_pairs, W):
    del similar_pairs
    sc1 = _sc_states_losses(
        states,
        dissimilar_pairs[:, 0], dissimilar_pairs[:, 1],
        ref_point_pairs[:, 0], ref_point_pairs[:, 1],
    )
    sc2 = _sc_pairdiff_losses(
        states, next_states,
        same_actions_pairs[:, 0], same_actions_pairs[:, 1],
    )
    part = _tc_dense(states.reshape(-1), next_states.reshape(-1), W)
    s1 = jnp.sum(sc1.reshape(_NW, 2, _LANES), axis=(0, 2))
    s2 = jnp.sum(sc2.reshape(_NW, 2, _LANES), axis=(0, 2))
    temp_coherence = jnp.sum(part[:, 0, 0]) / _N
    l1 = part[0, 0, 1]
    return (temp_coherence
            + s1[0] / _P
            + s2[0] / _P
            + s2[1] / _P
            + s1[1] / _R
            + _L1_COEFF * l1)
